# Initial kernel scaffold; baseline (speedup 1.0000x reference)
import functools

import jax
import jax.numpy as jnp
from jax import lax
from jax.experimental import pallas as pl
from jax.experimental.pallas import tpu as pltpu
from jax.experimental.pallas import tpu_sc as plsc

L = 577
S = 24
NU = 64
NW = 32
ROWS_PER_W = (L - 1) // NW


def _mesh():
    return plsc.VectorSubcoreMesh(
        core_axis_name="c", subcore_axis_name="s", num_cores=2, num_subcores=16
    )


@functools.partial(
    pl.kernel,
    out_type=jax.ShapeDtypeStruct((L, L, NU), jnp.float32),
    mesh=_mesh(),
    scratch_types=[
        pltpu.VMEM((29 * 47, NU), jnp.float32),
        pltpu.VMEM((1, L, NU), jnp.float32),
        pltpu.VMEM((30, NU), jnp.float32),
        pltpu.VMEM((47, NU), jnp.float32),
        pltpu.VMEM((1, NU), jnp.float32),
    ],
)
def _rp2d(tv_hbm, th_hbm, out_hbm, x_ref, row_ref, tv_ref, the_ref, e0_ref):
    cid = lax.axis_index("c")
    sid = lax.axis_index("s")
    wid = sid * 2 + cid

    pltpu.sync_copy(tv_hbm, tv_ref)
    pltpu.sync_copy(th_hbm.at[pl.ds(1, 29)], the_ref.at[pl.ds(9, 29)])
    pltpu.sync_copy(th_hbm.at[pl.ds(0, 1)], e0_ref)
    for d in range(9):
        for c in range(4):
            the_ref[d, pl.ds(c * 16, 16)] = the_ref[9, pl.ds(c * 16, 16)]
    for d in range(38, 47):
        for c in range(4):
            the_ref[d, pl.ds(c * 16, 16)] = the_ref[37, pl.ds(c * 16, 16)]
    for c in range(4):
        e0_ref[0, pl.ds(c * 16, 16)] = (
            e0_ref[0, pl.ds(c * 16, 16)] + tv_ref[0, pl.ds(c * 16, 16)]
        )

    def x_outer(fvi, _):
        tvc = [tv_ref[fvi + 1, pl.ds(c * 16, 16)] for c in range(4)]

        def x_inner(d, _):
            for c in range(4):
                x_ref[fvi * 47 + d, pl.ds(c * 16, 16)] = (
                    the_ref[d, pl.ds(c * 16, 16)] + tvc[c]
                )
            return 0

        lax.fori_loop(0, 47, x_inner, 0)
        return 0

    lax.fori_loop(0, 29, x_outer, 0)

    @pl.when(wid == 0)
    def _():
        def fill(j, _):
            for c in range(4):
                row_ref[0, j, pl.ds(c * 16, 16)] = e0_ref[0, pl.ds(c * 16, 16)]
            return 0

        lax.fori_loop(0, L, fill, 0)
        pltpu.sync_copy(row_ref, out_hbm.at[pl.ds(0, 1)])

    qb0 = jnp.where(wid >= S, 1, 0).astype(jnp.int32)
    qm0 = wid - S * qb0

    def row_body(t, carry):
        qb, qm = carry
        i = 1 + wid + NW * t
        off = 23 - qm

        def kb_body(kb, _):
            dlt = jnp.maximum(jnp.minimum(kb - qb, 14), -14)
            xrow0 = (dlt + 14) * 47 + off
            rrow0 = 1 + kb * S
            for u in range(S * 4):
                r = u // 4
                c0 = (u % 4) * 16
                row_ref[0, rrow0 + r, pl.ds(c0, 16)] = x_ref[
                    xrow0 + r, pl.ds(c0, 16)
                ]
            return 0

        lax.fori_loop(0, S, kb_body, 0)
        for c in range(4):
            row_ref[0, 0, pl.ds(c * 16, 16)] = e0_ref[0, pl.ds(c * 16, 16)]
        pltpu.sync_copy(row_ref, out_hbm.at[pl.ds(i, 1)])

        qb2 = qb + 1
        qm2 = qm + 8
        wrap = qm2 >= S
        qb2 = jnp.where(wrap, qb2 + 1, qb2)
        qm2 = jnp.where(wrap, qm2 - S, qm2)
        return (qb2, qm2)

    lax.fori_loop(0, ROWS_PER_W, row_body, (qb0, qm0))


def kernel(length_q, length_k, table_v, table_h):
    del length_q, length_k
    return _rp2d(table_v, table_h)


# speedup vs baseline: 3.4683x; 3.4683x over previous
import functools

import jax
import jax.numpy as jnp
from jax import lax
from jax.experimental import pallas as pl
from jax.experimental.pallas import tpu as pltpu
from jax.experimental.pallas import tpu_sc as plsc

L = 577
S = 24
NU = 64
NW = 32
ROWS_PER_W = (L - 1) // NW


def _mesh():
    return plsc.VectorSubcoreMesh(
        core_axis_name="c", subcore_axis_name="s", num_cores=2, num_subcores=16
    )


@functools.partial(
    pl.kernel,
    out_type=jax.ShapeDtypeStruct((L, L, NU), jnp.float32),
    mesh=_mesh(),
    scratch_types=[
        pltpu.VMEM((29 * 47, NU), jnp.float32),
        pltpu.VMEM((1, L, NU), jnp.float32),
        pltpu.VMEM((30, NU), jnp.float32),
        pltpu.VMEM((30, NU), jnp.float32),
        pltpu.VMEM((1, NU), jnp.float32),
    ],
    compiler_params=pltpu.CompilerParams(use_tc_tiling_on_sc=False),
)
def _rp2d(tv_hbm, th_hbm, out_hbm, x_ref, row_ref, tv_ref, th_ref, e0_ref):
    cid = lax.axis_index("c")
    sid = lax.axis_index("s")
    wid = sid * 2 + cid

    pltpu.sync_copy(tv_hbm, tv_ref)
    pltpu.sync_copy(th_hbm, th_ref)
    for c in range(4):
        e0_ref[0, pl.ds(c * 16, 16)] = (
            th_ref[0, pl.ds(c * 16, 16)] + tv_ref[0, pl.ds(c * 16, 16)]
        )

    def x_outer(fvi, _):
        tvc = [tv_ref[fvi + 1, pl.ds(c * 16, 16)] for c in range(4)]

        def x_inner(d, _):
            hrow = jnp.maximum(jnp.minimum(d - 23, 14), -14) + 15
            for c in range(4):
                x_ref[fvi * 47 + d, pl.ds(c * 16, 16)] = (
                    th_ref[hrow, pl.ds(c * 16, 16)] + tvc[c]
                )
            return 0

        lax.fori_loop(0, 47, x_inner, 0)
        return 0

    lax.fori_loop(0, 29, x_outer, 0)

    @pl.when(wid == 0)
    def _():
        def fill(j, _):
            for c in range(4):
                row_ref[0, j, pl.ds(c * 16, 16)] = e0_ref[0, pl.ds(c * 16, 16)]
            return 0

        lax.fori_loop(0, L, fill, 0)
        pltpu.sync_copy(row_ref, out_hbm.at[pl.ds(0, 1)])

    qb0 = jnp.where(wid >= S, 1, 0).astype(jnp.int32)
    qm0 = wid - S * qb0

    def row_body(t, carry):
        qb, qm = carry
        i = 1 + wid + NW * t
        off = 23 - qm

        def kb_body(kb, _):
            dlt = jnp.maximum(jnp.minimum(kb - qb, 14), -14)
            xrow0 = (dlt + 14) * 47 + off
            rrow0 = 1 + kb * S
            for u in range(S * 4):
                r = u // 4
                c0 = (u % 4) * 16
                row_ref[0, rrow0 + r, pl.ds(c0, 16)] = x_ref[
                    xrow0 + r, pl.ds(c0, 16)
                ]
            return 0

        lax.fori_loop(0, S, kb_body, 0)
        for c in range(4):
            row_ref[0, 0, pl.ds(c * 16, 16)] = e0_ref[0, pl.ds(c * 16, 16)]
        pltpu.sync_copy(row_ref, out_hbm.at[pl.ds(i, 1)])

        qb2 = qb + 1
        qm2 = qm + 8
        wrap = qm2 >= S
        qb2 = jnp.where(wrap, qb2 + 1, qb2)
        qm2 = jnp.where(wrap, qm2 - S, qm2)
        return (qb2, qm2)

    lax.fori_loop(0, ROWS_PER_W, row_body, (qb0, qm0))


def kernel(length_q, length_k, table_v, table_h):
    del length_q, length_k
    return _rp2d(table_v, table_h)


# speedup vs baseline: 4.7201x; 1.3609x over previous
import functools

import jax
import jax.numpy as jnp
from jax import lax
from jax.experimental import pallas as pl
from jax.experimental.pallas import tpu as pltpu
from jax.experimental.pallas import tpu_sc as plsc

L = 577
S = 24
NU = 64
NW = 32
ROWS_PER_W = (L - 1) // NW
JH = 288


def _mesh():
    return plsc.VectorSubcoreMesh(
        core_axis_name="c", subcore_axis_name="s", num_cores=2, num_subcores=16
    )


@functools.partial(
    pl.kernel,
    out_type=jax.ShapeDtypeStruct((L, L, NU), jnp.float32),
    mesh=_mesh(),
    scratch_types=[
        pltpu.VMEM((1, L, NU), jnp.float32),
        pltpu.VMEM((30, NU), jnp.float32),
        pltpu.VMEM((30, NU), jnp.float32),
        pltpu.VMEM((47, NU), jnp.float32),
        pltpu.VMEM((1, NU), jnp.float32),
        pltpu.SemaphoreType.DMA,
        pltpu.SemaphoreType.DMA,
    ],
    compiler_params=pltpu.CompilerParams(use_tc_tiling_on_sc=True),
)
def _rp2d(tv_hbm, th_hbm, out_hbm, row_ref, tv_ref, th_ref, the_ref, e0_ref,
          semA, semB):
    cid = lax.axis_index("c")
    sid = lax.axis_index("s")
    wid = sid * 2 + cid

    pltpu.sync_copy(tv_hbm, tv_ref)
    pltpu.sync_copy(th_hbm, th_ref)
    for d in range(47):
        hrow = min(max(d - 23, -14), 14) + 15
        for c in range(4):
            the_ref[d, pl.ds(c * 16, 16)] = th_ref[hrow, pl.ds(c * 16, 16)]
    for c in range(4):
        e0_ref[0, pl.ds(c * 16, 16)] = (
            th_ref[0, pl.ds(c * 16, 16)] + tv_ref[0, pl.ds(c * 16, 16)]
        )

    @pl.when(wid == 0)
    def _():
        def fill(j, _):
            for c in range(4):
                row_ref[0, j, pl.ds(c * 16, 16)] = e0_ref[0, pl.ds(c * 16, 16)]
            return 0

        lax.fori_loop(0, L, fill, 0)
        pltpu.sync_copy(row_ref, out_hbm.at[pl.ds(0, 1)])

    qb = jnp.where(wid >= S, 1, 0).astype(jnp.int32)
    qm = wid - S * qb

    def block_rows(kb, qb_, off_, r_lo, r_hi):
        dlt = jnp.maximum(jnp.minimum(kb - qb_, 14), -14)
        fv = dlt + 15
        tvc = [tv_ref[fv, pl.ds(c * 16, 16)] for c in range(4)]
        rrow0 = 1 + kb * S
        for r in range(r_lo, r_hi):
            for c in range(4):
                row_ref[0, rrow0 + r, pl.ds(c * 16, 16)] = (
                    the_ref[off_ + r, pl.ds(c * 16, 16)] + tvc[c]
                )

    srcA = row_ref.at[:, pl.ds(0, JH)]
    srcB = row_ref.at[:, pl.ds(JH, L - JH)]

    def row_body(t, carry):
        qb_, qm_ = carry
        i = 1 + wid + NW * t
        off = 23 - qm_

        @pl.when(t > 0)
        def _():
            pltpu.make_async_copy(
                srcA, out_hbm.at[pl.ds(1, 1), pl.ds(0, JH)], semA
            ).wait()

        def kbA(kb, _):
            block_rows(kb, qb_, off, 0, S)
            return 0

        lax.fori_loop(0, 11, kbA, 0)
        block_rows(11, qb_, off, 0, S - 1)
        for c in range(4):
            row_ref[0, 0, pl.ds(c * 16, 16)] = e0_ref[0, pl.ds(c * 16, 16)]
        pltpu.async_copy(srcA, out_hbm.at[pl.ds(i, 1), pl.ds(0, JH)], semA)

        @pl.when(t > 0)
        def _():
            pltpu.make_async_copy(
                srcB, out_hbm.at[pl.ds(1, 1), pl.ds(JH, L - JH)], semB
            ).wait()

        block_rows(11, qb_, off, S - 1, S)

        def kbB(kb, _):
            block_rows(kb, qb_, off, 0, S)
            return 0

        lax.fori_loop(11 + 1, S, kbB, 0)
        pltpu.async_copy(srcB, out_hbm.at[pl.ds(i, 1), pl.ds(JH, L - JH)], semB)

        qb2 = qb_ + 1
        qm2 = qm_ + 8
        wrap = qm2 >= S
        qb2 = jnp.where(wrap, qb2 + 1, qb2)
        qm2 = jnp.where(wrap, qm2 - S, qm2)
        return (qb2, qm2)

    lax.fori_loop(0, ROWS_PER_W, row_body, (qb, qm))

    pltpu.make_async_copy(srcA, out_hbm.at[pl.ds(1, 1), pl.ds(0, JH)], semA).wait()
    pltpu.make_async_copy(srcB, out_hbm.at[pl.ds(1, 1), pl.ds(JH, L - JH)], semB).wait()


def kernel(length_q, length_k, table_v, table_h):
    del length_q, length_k
    return _rp2d(table_v, table_h)
